# Initial kernel scaffold; baseline (speedup 1.0000x reference)
import jax
import jax.numpy as jnp
from jax import lax
from jax.experimental import pallas as pl
from jax.experimental.pallas import tpu as pltpu
from jax.experimental.pallas import tpu_sc as plsc

N = 10000
E = 320000
D = 128
HH = 64
EDGE_DIM = 125
EMB = 124

NC, NS = 2, 16
NW = NC * NS
CH = 128
NCHUNK = E // CH
CHUNK_REM = NCHUNK % NW
ROWS_PER_SUB = N // NS

EB = 2000
NB = 1000
ACC_W = 80

_sc_mesh = plsc.VectorSubcoreMesh(core_axis_name="c", subcore_axis_name="s")


def _node_mm_body(x_ref, w_ref, xl_ref, xr_ref):
    o = jnp.dot(x_ref[...], w_ref[...], preferred_element_type=jnp.float32)
    xl_ref[...] = o[:, :HH]
    xr_ref[...] = o[:, HH:]


def _node_mm(x, w_node):
    return pl.pallas_call(
        _node_mm_body,
        grid=(N // NB,),
        in_specs=[
            pl.BlockSpec((NB, D), lambda i: (i, 0)),
            pl.BlockSpec((D, 2 * HH), lambda i: (0, 0)),
        ],
        out_specs=[pl.BlockSpec((NB, HH), lambda i: (i, 0))] * 2,
        out_shape=[jax.ShapeDtypeStruct((N, HH), jnp.float32)] * 2,
    )(x, w_node)


def _edge1_body(ea_ref, gl_ref, gr_ref, we_ref, w1c_ref, att_ref,
                wvp_ref, qc_ref):
    ea = ea_ref[...]
    gl = gl_ref[...]
    m = gl + gr_ref[...] + jnp.dot(ea, we_ref[...],
                                   preferred_element_type=jnp.float32)
    m = jnp.where(m >= 0, m, 0.2 * m)
    am = m * att_ref[...]
    p0 = jnp.exp(jnp.sum(am[:, :32], axis=1, keepdims=True))
    p1 = jnp.exp(jnp.sum(am[:, 32:], axis=1, keepdims=True))
    wvp_ref[...] = jnp.concatenate(
        [gl[:, :32] * p0, gl[:, 32:] * p1, p0, p1,
         jnp.zeros((EB, ACC_W - HH - 2), jnp.float32)], axis=1)
    qc_ref[...] = jnp.dot(ea[:, :EMB], w1c_ref[...],
                          preferred_element_type=jnp.float32)


def _edge1(edge_attr, gl, gr, we, w1c, att_row):
    return pl.pallas_call(
        _edge1_body,
        grid=(E // EB,),
        in_specs=[
            pl.BlockSpec((EB, EDGE_DIM), lambda i: (i, 0)),
            pl.BlockSpec((EB, HH), lambda i: (i, 0)),
            pl.BlockSpec((EB, HH), lambda i: (i, 0)),
            pl.BlockSpec((EDGE_DIM, HH), lambda i: (0, 0)),
            pl.BlockSpec((EMB, HH), lambda i: (0, 0)),
            pl.BlockSpec((1, HH), lambda i: (0, 0)),
        ],
        out_specs=[
            pl.BlockSpec((EB, ACC_W), lambda i: (i, 0)),
            pl.BlockSpec((EB, HH), lambda i: (i, 0)),
        ],
        out_shape=[
            jax.ShapeDtypeStruct((E, ACC_W), jnp.float32),
            jax.ShapeDtypeStruct((E, HH), jnp.float32),
        ],
    )(edge_attr, gl, gr, we, w1c, att_row)


def _node2_body(a0_ref, a1_ref, bias_ref, w1a_ref, w1b_ref, g1_ref, g2_ref):
    s = a0_ref[...] + a1_ref[...]
    v = s[:, :HH]
    d0 = s[:, HH:HH + 1] + 1e-16
    d1 = s[:, HH + 1:HH + 2] + 1e-16
    h = jnp.concatenate([v[:, :32] / d0, v[:, 32:] / d1], axis=1)
    h = jnp.maximum(h + bias_ref[...], 0.0)
    g1_ref[...] = jnp.dot(h, w1a_ref[...], preferred_element_type=jnp.float32)
    g2_ref[...] = jnp.dot(h, w1b_ref[...], preferred_element_type=jnp.float32)


def _node2(acc0, acc1, bias_row, w1a, w1b):
    return pl.pallas_call(
        _node2_body,
        grid=(N // NB,),
        in_specs=[
            pl.BlockSpec((NB, ACC_W), lambda i: (i, 0)),
            pl.BlockSpec((NB, ACC_W), lambda i: (i, 0)),
            pl.BlockSpec((1, HH), lambda i: (0, 0)),
            pl.BlockSpec((HH, HH), lambda i: (0, 0)),
            pl.BlockSpec((HH, HH), lambda i: (0, 0)),
        ],
        out_specs=[pl.BlockSpec((NB, HH), lambda i: (i, 0))] * 2,
        out_shape=[jax.ShapeDtypeStruct((N, HH), jnp.float32)] * 2,
    )(acc0, acc1, bias_row, w1a, w1b)


def _edge2_body(g1_ref, g2_ref, qc_ref, b1_ref, w2t_ref, b2_ref, out_ref):
    hcls = jnp.maximum(g1_ref[...] + g2_ref[...] + qc_ref[...] + b1_ref[...],
                       0.0)
    w2t = w2t_ref[...]
    b2 = b2_ref[...]
    l0 = jnp.sum(hcls * w2t[0:1, :], axis=1, keepdims=True) + b2[:, 0:1]
    l1 = jnp.sum(hcls * w2t[1:2, :], axis=1, keepdims=True) + b2[:, 1:2]
    l2 = jnp.sum(hcls * w2t[2:3, :], axis=1, keepdims=True) + b2[:, 2:3]
    mx = jnp.maximum(jnp.maximum(l0, l1), l2)
    e0 = jnp.exp(l0 - mx)
    e1 = jnp.exp(l1 - mx)
    e2 = jnp.exp(l2 - mx)
    out_ref[...] = (0.5 * e1 + e2) / (e0 + e1 + e2)


def _edge2(gg1, gg2, qc, b1_row, w2t, b2_row):
    return pl.pallas_call(
        _edge2_body,
        grid=(E // EB,),
        in_specs=[
            pl.BlockSpec((EB, HH), lambda i: (i, 0)),
            pl.BlockSpec((EB, HH), lambda i: (i, 0)),
            pl.BlockSpec((EB, HH), lambda i: (i, 0)),
            pl.BlockSpec((1, HH), lambda i: (0, 0)),
            pl.BlockSpec((3, HH), lambda i: (0, 0)),
            pl.BlockSpec((1, 3), lambda i: (0, 0)),
        ],
        out_specs=pl.BlockSpec((EB, 1), lambda i: (i, 0)),
        out_shape=jax.ShapeDtypeStruct((E, 1), jnp.float32),
    )(gg1, gg2, qc, b1_row, w2t, b2_row)


def _sc_gather_body(tab_a, idx_a, tab_b, idx_b, out_a, out_b,
                    idx_v, rows_v, sem):
    w = lax.axis_index("s") * NC + lax.axis_index("c")
    n = jnp.where(w < CHUNK_REM, NCHUNK // NW + 1, NCHUNK // NW)

    def body(i, carry):
        ch = w + i * NW
        pltpu.sync_copy(idx_a.at[ch], idx_v)
        pltpu.async_copy(tab_a.at[idx_v], rows_v, sem).wait()
        pltpu.sync_copy(rows_v, out_a.at[pl.ds(ch * CH, CH)])
        pltpu.sync_copy(idx_b.at[ch], idx_v)
        pltpu.async_copy(tab_b.at[idx_v], rows_v, sem).wait()
        pltpu.sync_copy(rows_v, out_b.at[pl.ds(ch * CH, CH)])
        return carry

    lax.fori_loop(0, n, body, 0)


_sc_gather = pl.kernel(
    _sc_gather_body,
    out_type=[jax.ShapeDtypeStruct((E, HH), jnp.float32)] * 2,
    mesh=_sc_mesh,
    scratch_types=[
        pltpu.VMEM((CH,), jnp.int32),
        pltpu.VMEM((CH, HH), jnp.float32),
        pltpu.SemaphoreType.DMA,
    ],
)


def _sc_scatter_body(wvp, idx2d, zrows, acc_out, acc_sh, idx_v, rows_v):
    c = lax.axis_index("c")
    s = lax.axis_index("s")
    w = s * NC + c
    n = jnp.where(w < CHUNK_REM, NCHUNK // NW + 1, NCHUNK // NW)

    pltpu.sync_copy(zrows.at[pl.ds(s * ROWS_PER_SUB, ROWS_PER_SUB)],
                    acc_sh.at[pl.ds(s * ROWS_PER_SUB, ROWS_PER_SUB)])
    plsc.subcore_barrier()

    def body(i, carry):
        ch = w + i * NW
        pltpu.sync_copy(idx2d.at[ch], idx_v)
        pltpu.sync_copy(wvp.at[pl.ds(ch * CH, CH)], rows_v)
        pltpu.sync_copy(rows_v, acc_sh.at[idx_v], add=True)
        return carry

    lax.fori_loop(0, n, body, 0)
    plsc.subcore_barrier()
    pltpu.sync_copy(acc_sh.at[pl.ds(s * ROWS_PER_SUB, ROWS_PER_SUB)],
                    acc_out.at[c, pl.ds(s * ROWS_PER_SUB, ROWS_PER_SUB)])


_sc_scatter = pl.kernel(
    _sc_scatter_body,
    out_type=jax.ShapeDtypeStruct((NC, N, ACC_W), jnp.float32),
    mesh=_sc_mesh,
    scratch_types=[
        pltpu.VMEM_SHARED((N, ACC_W), jnp.float32),
        pltpu.VMEM((CH,), jnp.int32),
        pltpu.VMEM((CH, ACC_W), jnp.float32),
    ],
)


def kernel(x, edge_index, edge_attributes, Wl, Wr, We, att, bias,
           W1, b1, W2, b2):
    src2d = edge_index[0].astype(jnp.int32).reshape(NCHUNK, CH)
    dst2d = edge_index[1].astype(jnp.int32).reshape(NCHUNK, CH)
    w_node = jnp.concatenate([Wl, Wr], axis=1)
    att_row = att.reshape(1, HH)
    w1a = W1[:HH]
    w1b = W1[HH:2 * HH]
    w1c = W1[2 * HH:]
    bias_row = bias.reshape(1, HH)
    b1_row = b1.reshape(1, HH)
    w2t = W2.T
    b2_row = b2.reshape(1, 3)
    zrows = jnp.zeros((N, ACC_W), jnp.float32)

    xl, xr = _node_mm(x, w_node)
    gl, gr = _sc_gather(xl, src2d, xr, dst2d)
    wvp, qc = _edge1(edge_attributes, gl, gr, We, w1c, att_row)
    acc = _sc_scatter(wvp, dst2d, zrows)
    g1, g2 = _node2(acc[0], acc[1], bias_row, w1a, w1b)
    gg1, gg2 = _sc_gather(g1, src2d, g2, dst2d)
    trust = _edge2(gg1, gg2, qc, b1_row, w2t, b2_row)
    return trust.reshape(E)


# speedup vs baseline: 15.8172x; 15.8172x over previous
import functools

import jax
import jax.numpy as jnp
from jax import lax
from jax.experimental import pallas as pl
from jax.experimental.pallas import tpu as pltpu
from jax.experimental.pallas import tpu_sc as plsc

N = 10000
E = 320000
D = 128
HH = 64
EDGE_DIM = 125
EMB = 124

NC, NS = 2, 16
NW = NC * NS
CH = 128
NCHUNK = E // CH
CHUNK_REM = NCHUNK % NW
ROWS_PER_SUB = N // NS

EB = 2000
NB = 1000
ACC_W = 80


def _node_mm_body(x_ref, w_ref, xl_ref, xr_ref):
    o = jnp.dot(x_ref[...], w_ref[...], preferred_element_type=jnp.float32)
    xl_ref[...] = o[:, :HH]
    xr_ref[...] = o[:, HH:]


def _node_mm(x, w_node):
    return pl.pallas_call(
        _node_mm_body,
        grid=(N // NB,),
        in_specs=[
            pl.BlockSpec((NB, D), lambda i: (i, 0)),
            pl.BlockSpec((D, 2 * HH), lambda i: (0, 0)),
        ],
        out_specs=[pl.BlockSpec((NB, HH), lambda i: (i, 0))] * 2,
        out_shape=[jax.ShapeDtypeStruct((N, HH), jnp.float32)] * 2,
    )(x, w_node)


def _edge1_body(ea_ref, gl_ref, gr_ref, we_ref, w1c_ref, att_ref,
                wvp_ref, qc_ref):
    ea = ea_ref[...]
    gl = gl_ref[...]
    m = gl + gr_ref[...] + jnp.dot(ea, we_ref[...],
                                   preferred_element_type=jnp.float32)
    m = jnp.where(m >= 0, m, 0.2 * m)
    am = m * att_ref[...]
    p0 = jnp.exp(jnp.sum(am[:, :32], axis=1, keepdims=True))
    p1 = jnp.exp(jnp.sum(am[:, 32:], axis=1, keepdims=True))
    wvp_ref[...] = jnp.concatenate(
        [gl[:, :32] * p0, gl[:, 32:] * p1, p0, p1,
         jnp.zeros((EB, ACC_W - HH - 2), jnp.float32)], axis=1)
    qc_ref[...] = jnp.dot(ea[:, :EMB], w1c_ref[...],
                          preferred_element_type=jnp.float32)


def _edge1(edge_attr, gl, gr, we, w1c, att_row):
    return pl.pallas_call(
        _edge1_body,
        grid=(E // EB,),
        in_specs=[
            pl.BlockSpec((EB, EDGE_DIM), lambda i: (i, 0)),
            pl.BlockSpec((EB, HH), lambda i: (i, 0)),
            pl.BlockSpec((EB, HH), lambda i: (i, 0)),
            pl.BlockSpec((EDGE_DIM, HH), lambda i: (0, 0)),
            pl.BlockSpec((EMB, HH), lambda i: (0, 0)),
            pl.BlockSpec((1, HH), lambda i: (0, 0)),
        ],
        out_specs=[
            pl.BlockSpec((EB, ACC_W), lambda i: (i, 0)),
            pl.BlockSpec((EB, HH), lambda i: (i, 0)),
        ],
        out_shape=[
            jax.ShapeDtypeStruct((E, ACC_W), jnp.float32),
            jax.ShapeDtypeStruct((E, HH), jnp.float32),
        ],
    )(edge_attr, gl, gr, we, w1c, att_row)


def _node2_body(a0_ref, a1_ref, bias_ref, w1a_ref, w1b_ref, g1_ref, g2_ref):
    s = a0_ref[...] + a1_ref[...]
    v = s[:, :HH]
    d0 = s[:, HH:HH + 1] + 1e-16
    d1 = s[:, HH + 1:HH + 2] + 1e-16
    h = jnp.concatenate([v[:, :32] / d0, v[:, 32:] / d1], axis=1)
    h = jnp.maximum(h + bias_ref[...], 0.0)
    g1_ref[...] = jnp.dot(h, w1a_ref[...], preferred_element_type=jnp.float32)
    g2_ref[...] = jnp.dot(h, w1b_ref[...], preferred_element_type=jnp.float32)


def _node2(acc0, acc1, bias_row, w1a, w1b):
    return pl.pallas_call(
        _node2_body,
        grid=(N // NB,),
        in_specs=[
            pl.BlockSpec((NB, ACC_W), lambda i: (i, 0)),
            pl.BlockSpec((NB, ACC_W), lambda i: (i, 0)),
            pl.BlockSpec((1, HH), lambda i: (0, 0)),
            pl.BlockSpec((HH, HH), lambda i: (0, 0)),
            pl.BlockSpec((HH, HH), lambda i: (0, 0)),
        ],
        out_specs=[pl.BlockSpec((NB, HH), lambda i: (i, 0))] * 2,
        out_shape=[jax.ShapeDtypeStruct((N, HH), jnp.float32)] * 2,
    )(acc0, acc1, bias_row, w1a, w1b)


def _edge2_body(g1_ref, g2_ref, qc_ref, b1_ref, w2t_ref, b2_ref, out_ref):
    hcls = jnp.maximum(g1_ref[...] + g2_ref[...] + qc_ref[...] + b1_ref[...],
                       0.0)
    w2t = w2t_ref[...]
    b2 = b2_ref[...]
    l0 = jnp.sum(hcls * w2t[0:1, :], axis=1, keepdims=True) + b2[:, 0:1]
    l1 = jnp.sum(hcls * w2t[1:2, :], axis=1, keepdims=True) + b2[:, 1:2]
    l2 = jnp.sum(hcls * w2t[2:3, :], axis=1, keepdims=True) + b2[:, 2:3]
    mx = jnp.maximum(jnp.maximum(l0, l1), l2)
    e0 = jnp.exp(l0 - mx)
    e1 = jnp.exp(l1 - mx)
    e2 = jnp.exp(l2 - mx)
    out_ref[...] = (0.5 * e1 + e2) / (e0 + e1 + e2)


def _edge2(gg1, gg2, qc, b1_row, w2t, b2_row):
    return pl.pallas_call(
        _edge2_body,
        grid=(E // EB,),
        in_specs=[
            pl.BlockSpec((EB, HH), lambda i: (i, 0)),
            pl.BlockSpec((EB, HH), lambda i: (i, 0)),
            pl.BlockSpec((EB, HH), lambda i: (i, 0)),
            pl.BlockSpec((1, HH), lambda i: (0, 0)),
            pl.BlockSpec((3, HH), lambda i: (0, 0)),
            pl.BlockSpec((1, 3), lambda i: (0, 0)),
        ],
        out_specs=pl.BlockSpec((EB, 1), lambda i: (i, 0)),
        out_shape=jax.ShapeDtypeStruct((E, 1), jnp.float32),
    )(gg1, gg2, qc, b1_row, w2t, b2_row)


def _sc_gather_body(tab_a, idx_a, tab_b, idx_b, out_a, out_b,
                    idx_v, rows_v, sem):
    w = lax.axis_index("s") * NC + lax.axis_index("c")
    n = jnp.where(w < CHUNK_REM, NCHUNK // NW + 1, NCHUNK // NW)

    def body(i, carry):
        ch = w + i * NW
        pltpu.sync_copy(idx_a.at[ch], idx_v)
        pltpu.async_copy(tab_a.at[idx_v], rows_v, sem).wait()
        pltpu.sync_copy(rows_v, out_a.at[pl.ds(ch * CH, CH)])
        pltpu.sync_copy(idx_b.at[ch], idx_v)
        pltpu.async_copy(tab_b.at[idx_v], rows_v, sem).wait()
        pltpu.sync_copy(rows_v, out_b.at[pl.ds(ch * CH, CH)])
        return carry

    lax.fori_loop(0, n, body, 0)


def _sc_scatter_body(wvp, idx2d, zrows, acc_out, acc_sh, idx_v, rows_v):
    c = lax.axis_index("c")
    s = lax.axis_index("s")
    w = s * NC + c
    n = jnp.where(w < CHUNK_REM, NCHUNK // NW + 1, NCHUNK // NW)

    pltpu.sync_copy(zrows.at[pl.ds(s * ROWS_PER_SUB, ROWS_PER_SUB)],
                    acc_sh.at[pl.ds(s * ROWS_PER_SUB, ROWS_PER_SUB)])
    plsc.subcore_barrier()

    def body(i, carry):
        ch = w + i * NW
        pltpu.sync_copy(idx2d.at[ch], idx_v)
        pltpu.sync_copy(wvp.at[pl.ds(ch * CH, CH)], rows_v)
        pltpu.sync_copy(rows_v, acc_sh.at[idx_v], add=True)
        return carry

    lax.fori_loop(0, n, body, 0)
    plsc.subcore_barrier()
    pltpu.sync_copy(acc_sh.at[pl.ds(s * ROWS_PER_SUB, ROWS_PER_SUB)],
                    acc_out.at[c, pl.ds(s * ROWS_PER_SUB, ROWS_PER_SUB)])


@functools.cache
def _sc_kernels():
    mesh = plsc.VectorSubcoreMesh(core_axis_name="c", subcore_axis_name="s",
                                  num_cores=NC, num_subcores=NS)
    params = pltpu.CompilerParams(use_tc_tiling_on_sc=False)
    gather = pl.kernel(
        _sc_gather_body,
        out_type=[jax.ShapeDtypeStruct((E, HH), jnp.float32)] * 2,
        mesh=mesh,
        compiler_params=params,
        scratch_types=[
            pltpu.VMEM((CH,), jnp.int32),
            pltpu.VMEM((CH, HH), jnp.float32),
            pltpu.SemaphoreType.DMA,
        ],
    )
    scatter = pl.kernel(
        _sc_scatter_body,
        out_type=jax.ShapeDtypeStruct((NC, N, ACC_W), jnp.float32),
        mesh=mesh,
        compiler_params=params,
        scratch_types=[
            pltpu.VMEM_SHARED((N, ACC_W), jnp.float32),
            pltpu.VMEM((CH,), jnp.int32),
            pltpu.VMEM((CH, ACC_W), jnp.float32),
        ],
    )
    return gather, scatter


def kernel(x, edge_index, edge_attributes, Wl, Wr, We, att, bias,
           W1, b1, W2, b2):
    src2d = edge_index[0].astype(jnp.int32).reshape(NCHUNK, CH)
    dst2d = edge_index[1].astype(jnp.int32).reshape(NCHUNK, CH)
    w_node = jnp.concatenate([Wl, Wr], axis=1)
    att_row = att.reshape(1, HH)
    w1a = W1[:HH]
    w1b = W1[HH:2 * HH]
    w1c = W1[2 * HH:]
    bias_row = bias.reshape(1, HH)
    b1_row = b1.reshape(1, HH)
    w2t = W2.T
    b2_row = b2.reshape(1, 3)
    zrows = jnp.zeros((N, ACC_W), jnp.float32)

    sc_gather, sc_scatter = _sc_kernels()
    xl, xr = _node_mm(x, w_node)
    gl, gr = sc_gather(xl, src2d, xr, dst2d)
    wvp, qc = _edge1(edge_attributes, gl, gr, We, w1c, att_row)
    acc = sc_scatter(wvp, dst2d, zrows)
    g1, g2 = _node2(acc[0], acc[1], bias_row, w1a, w1b)
    gg1, gg2 = sc_gather(g1, src2d, g2, dst2d)
    trust = _edge2(gg1, gg2, qc, b1_row, w2t, b2_row)
    return trust.reshape(E)


# speedup vs baseline: 19.1680x; 1.2118x over previous
import functools

import jax
import jax.numpy as jnp
from jax import lax
from jax.experimental import pallas as pl
from jax.experimental.pallas import tpu as pltpu
from jax.experimental.pallas import tpu_sc as plsc

N = 10000
E = 320000
D = 128
HH = 64
EDGE_DIM = 125
EMB = 124

NC, NS = 2, 16
NW = NC * NS
CH = 128
NCHUNK = E // CH
CPW = 80
PCH = NW * CPW
GRP = 2
NGRP = CPW // GRP
ROWS_PER_SUB = N // NS

EB = 2000
NB = 1000
ACC_W = 80


def _node_mm_body(x_ref, w_ref, xl_ref, xr_ref):
    o = jnp.dot(x_ref[...], w_ref[...], preferred_element_type=jnp.float32)
    xl_ref[...] = o[:, :HH]
    xr_ref[...] = o[:, HH:]


def _node_mm(x, w_node):
    return pl.pallas_call(
        _node_mm_body,
        grid=(N // NB,),
        in_specs=[
            pl.BlockSpec((NB, D), lambda i: (i, 0)),
            pl.BlockSpec((D, 2 * HH), lambda i: (0, 0)),
        ],
        out_specs=[pl.BlockSpec((NB, HH), lambda i: (i, 0))] * 2,
        out_shape=[jax.ShapeDtypeStruct((N, HH), jnp.float32)] * 2,
    )(x, w_node)


def _edge1_body(ea_ref, gl_ref, gr_ref, we_ref, w1c_ref, att_ref,
                wvp_ref, qc_ref):
    ea = ea_ref[...]
    gl = gl_ref[...]
    m = gl + gr_ref[...] + jnp.dot(ea, we_ref[...],
                                   preferred_element_type=jnp.float32)
    m = jnp.where(m >= 0, m, 0.2 * m)
    am = m * att_ref[...]
    p0 = jnp.exp(jnp.sum(am[:, :32], axis=1, keepdims=True))
    p1 = jnp.exp(jnp.sum(am[:, 32:], axis=1, keepdims=True))
    wvp_ref[...] = jnp.concatenate(
        [gl[:, :32] * p0, gl[:, 32:] * p1, p0, p1,
         jnp.zeros((EB, ACC_W - HH - 2), jnp.float32)], axis=1)
    qc_ref[...] = jnp.dot(ea[:, :EMB], w1c_ref[...],
                          preferred_element_type=jnp.float32)


def _edge1(edge_attr, gl, gr, we, w1c, att_row):
    return pl.pallas_call(
        _edge1_body,
        grid=(E // EB,),
        in_specs=[
            pl.BlockSpec((EB, EDGE_DIM), lambda i: (i, 0)),
            pl.BlockSpec((EB, HH), lambda i: (i, 0)),
            pl.BlockSpec((EB, HH), lambda i: (i, 0)),
            pl.BlockSpec((EDGE_DIM, HH), lambda i: (0, 0)),
            pl.BlockSpec((EMB, HH), lambda i: (0, 0)),
            pl.BlockSpec((1, HH), lambda i: (0, 0)),
        ],
        out_specs=[
            pl.BlockSpec((EB, ACC_W), lambda i: (i, 0)),
            pl.BlockSpec((EB, HH), lambda i: (i, 0)),
        ],
        out_shape=[
            jax.ShapeDtypeStruct((E, ACC_W), jnp.float32),
            jax.ShapeDtypeStruct((E, HH), jnp.float32),
        ],
    )(edge_attr, gl, gr, we, w1c, att_row)


def _node2_body(a0_ref, a1_ref, bias_ref, w1a_ref, w1b_ref, g1_ref, g2_ref):
    s = a0_ref[...] + a1_ref[...]
    v = s[:, :HH]
    d0 = s[:, HH:HH + 1] + 1e-16
    d1 = s[:, HH + 1:HH + 2] + 1e-16
    h = jnp.concatenate([v[:, :32] / d0, v[:, 32:] / d1], axis=1)
    h = jnp.maximum(h + bias_ref[...], 0.0)
    g1_ref[...] = jnp.dot(h, w1a_ref[...], preferred_element_type=jnp.float32)
    g2_ref[...] = jnp.dot(h, w1b_ref[...], preferred_element_type=jnp.float32)


def _node2(acc0, acc1, bias_row, w1a, w1b):
    return pl.pallas_call(
        _node2_body,
        grid=(N // NB,),
        in_specs=[
            pl.BlockSpec((NB, ACC_W), lambda i: (i, 0)),
            pl.BlockSpec((NB, ACC_W), lambda i: (i, 0)),
            pl.BlockSpec((1, HH), lambda i: (0, 0)),
            pl.BlockSpec((HH, HH), lambda i: (0, 0)),
            pl.BlockSpec((HH, HH), lambda i: (0, 0)),
        ],
        out_specs=[pl.BlockSpec((NB, HH), lambda i: (i, 0))] * 2,
        out_shape=[jax.ShapeDtypeStruct((N, HH), jnp.float32)] * 2,
    )(acc0, acc1, bias_row, w1a, w1b)


def _edge2_body(g1_ref, g2_ref, qc_ref, b1_ref, w2t_ref, b2_ref, out_ref):
    hcls = jnp.maximum(g1_ref[...] + g2_ref[...] + qc_ref[...] + b1_ref[...],
                       0.0)
    w2t = w2t_ref[...]
    b2 = b2_ref[...]
    l0 = jnp.sum(hcls * w2t[0:1, :], axis=1, keepdims=True) + b2[:, 0:1]
    l1 = jnp.sum(hcls * w2t[1:2, :], axis=1, keepdims=True) + b2[:, 1:2]
    l2 = jnp.sum(hcls * w2t[2:3, :], axis=1, keepdims=True) + b2[:, 2:3]
    mx = jnp.maximum(jnp.maximum(l0, l1), l2)
    e0 = jnp.exp(l0 - mx)
    e1 = jnp.exp(l1 - mx)
    e2 = jnp.exp(l2 - mx)
    out_ref[...] = (0.5 * e1 + e2) / (e0 + e1 + e2)


def _edge2(gg1, gg2, qc, b1_row, w2t, b2_row):
    return pl.pallas_call(
        _edge2_body,
        grid=(E // EB,),
        in_specs=[
            pl.BlockSpec((EB, HH), lambda i: (i, 0)),
            pl.BlockSpec((EB, HH), lambda i: (i, 0)),
            pl.BlockSpec((EB, HH), lambda i: (i, 0)),
            pl.BlockSpec((1, HH), lambda i: (0, 0)),
            pl.BlockSpec((3, HH), lambda i: (0, 0)),
            pl.BlockSpec((1, 3), lambda i: (0, 0)),
        ],
        out_specs=pl.BlockSpec((EB, 1), lambda i: (i, 0)),
        out_shape=jax.ShapeDtypeStruct((E, 1), jnp.float32),
    )(gg1, gg2, qc, b1_row, w2t, b2_row)


def _sc_gather_body(tab_a, idx_a, tab_b, idx_b, out_a, out_b,
                    ia, ib, ra, rb, sg, sw0, sw1):
    w = lax.axis_index("s") * NC + lax.axis_index("c")
    base = w * CPW
    pltpu.sync_copy(idx_a.at[pl.ds(base, CPW)], ia)
    pltpu.sync_copy(idx_b.at[pl.ds(base, CPW)], ib)
    sws = (sw0, sw1)

    def real(g, k):
        return base + g * GRP + k < NCHUNK

    def g_descs(par, j, k):
        da = pltpu.make_async_copy(tab_a.at[ia.at[j]],
                                   ra.at[par, pl.ds(k * CH, CH)], sg)
        db = pltpu.make_async_copy(tab_b.at[ib.at[j]],
                                   rb.at[par, pl.ds(k * CH, CH)], sg)
        return da, db

    def wb_descs(par, g, k):
        row0 = (base + g * GRP + k) * CH
        da = pltpu.make_async_copy(ra.at[par, pl.ds(k * CH, CH)],
                                   out_a.at[pl.ds(row0, CH)], sws[par])
        db = pltpu.make_async_copy(rb.at[par, pl.ds(k * CH, CH)],
                                   out_b.at[pl.ds(row0, CH)], sws[par])
        return da, db

    def body(t, carry):
        for par in (0, 1):
            g = t * 2 + par
            for k in range(GRP):
                @pl.when((t > 0) & real(g - 2, k))
                def _(par=par, g=g, k=k):
                    da, db = wb_descs(par, g - 2, k)
                    da.wait()
                    db.wait()

                @pl.when(real(g, k))
                def _(par=par, g=g, k=k):
                    da, db = g_descs(par, g * GRP + k, k)
                    da.start()
                    db.start()
        for par in (0, 1):
            g = t * 2 + par
            for k in range(GRP):
                @pl.when(real(g, k))
                def _(par=par, g=g, k=k):
                    da, db = g_descs(par, g * GRP + k, k)
                    da.wait()
                    db.wait()
                    wa, wb_ = wb_descs(par, g, k)
                    wa.start()
                    wb_.start()
        return carry

    lax.fori_loop(0, NGRP // 2, body, 0)
    for par in (0, 1):
        g = NGRP - 2 + par
        for k in range(GRP):
            @pl.when(real(g, k))
            def _(par=par, g=g, k=k):
                da, db = wb_descs(par, g, k)
                da.wait()
                db.wait()


def _sc_scatter_body(wvp, idx2d, zrows, acc_out, acc_sh, id2, wb, sl0, sl1, ss):
    c = lax.axis_index("c")
    s = lax.axis_index("s")
    w = s * NC + c
    base = w * CPW
    pltpu.sync_copy(idx2d.at[pl.ds(base, CPW)], id2)
    pltpu.sync_copy(zrows.at[pl.ds(s * ROWS_PER_SUB, ROWS_PER_SUB)],
                    acc_sh.at[pl.ds(s * ROWS_PER_SUB, ROWS_PER_SUB)])
    plsc.subcore_barrier()
    sls = (sl0, sl1)

    def real(g, k):
        return base + g * GRP + k < NCHUNK

    def l_desc(par, g, k):
        row0 = (base + g * GRP + k) * CH
        return pltpu.make_async_copy(wvp.at[pl.ds(row0, CH)],
                                     wb.at[par, pl.ds(k * CH, CH)], sls[par])

    def s_desc(par, j, k):
        return pltpu.make_async_copy(wb.at[par, pl.ds(k * CH, CH)],
                                     acc_sh.at[id2.at[j]], ss)

    def body(t, carry):
        for par in (0, 1):
            g = t * 2 + par
            for k in range(GRP):
                @pl.when(real(g, k))
                def _(par=par, g=g, k=k):
                    l_desc(par, g, k).start()
        for par in (0, 1):
            g = t * 2 + par
            for k in range(GRP):
                @pl.when(real(g, k))
                def _(par=par, g=g, k=k):
                    l_desc(par, g, k).wait()
                    s_desc(par, g * GRP + k, k).start(add=True)
            for k in range(GRP):
                @pl.when(real(g, k))
                def _(par=par, g=g, k=k):
                    s_desc(par, g * GRP + k, k).wait()
        return carry

    lax.fori_loop(0, NGRP // 2, body, 0)
    plsc.subcore_barrier()
    pltpu.sync_copy(acc_sh.at[pl.ds(s * ROWS_PER_SUB, ROWS_PER_SUB)],
                    acc_out.at[c, pl.ds(s * ROWS_PER_SUB, ROWS_PER_SUB)])


@functools.cache
def _sc_kernels():
    mesh = plsc.VectorSubcoreMesh(core_axis_name="c", subcore_axis_name="s",
                                  num_cores=NC, num_subcores=NS)
    params = pltpu.CompilerParams(use_tc_tiling_on_sc=False)
    gather = pl.kernel(
        _sc_gather_body,
        out_type=[jax.ShapeDtypeStruct((E, HH), jnp.float32)] * 2,
        mesh=mesh,
        compiler_params=params,
        scratch_types=[
            pltpu.VMEM((CPW, CH), jnp.int32),
            pltpu.VMEM((CPW, CH), jnp.int32),
            pltpu.VMEM((2, GRP * CH, HH), jnp.float32),
            pltpu.VMEM((2, GRP * CH, HH), jnp.float32),
            pltpu.SemaphoreType.DMA,
            pltpu.SemaphoreType.DMA,
            pltpu.SemaphoreType.DMA,
        ],
    )
    scatter = pl.kernel(
        _sc_scatter_body,
        out_type=jax.ShapeDtypeStruct((NC, N, ACC_W), jnp.float32),
        mesh=mesh,
        compiler_params=params,
        scratch_types=[
            pltpu.VMEM_SHARED((N, ACC_W), jnp.float32),
            pltpu.VMEM((CPW, CH), jnp.int32),
            pltpu.VMEM((2, GRP * CH, ACC_W), jnp.float32),
            pltpu.SemaphoreType.DMA,
            pltpu.SemaphoreType.DMA,
            pltpu.SemaphoreType.DMA,
        ],
    )
    return gather, scatter


def kernel(x, edge_index, edge_attributes, Wl, Wr, We, att, bias,
           W1, b1, W2, b2):
    pad = jnp.zeros(((PCH - NCHUNK) * CH,), jnp.int32)
    src2d = jnp.concatenate(
        [edge_index[0].astype(jnp.int32), pad]).reshape(PCH, CH)
    dst2d = jnp.concatenate(
        [edge_index[1].astype(jnp.int32), pad]).reshape(PCH, CH)
    w_node = jnp.concatenate([Wl, Wr], axis=1)
    att_row = att.reshape(1, HH)
    w1a = W1[:HH]
    w1b = W1[HH:2 * HH]
    w1c = W1[2 * HH:]
    bias_row = bias.reshape(1, HH)
    b1_row = b1.reshape(1, HH)
    w2t = W2.T
    b2_row = b2.reshape(1, 3)
    zrows = jnp.zeros((N, ACC_W), jnp.float32)

    sc_gather, sc_scatter = _sc_kernels()
    xl, xr = _node_mm(x, w_node)
    gl, gr = sc_gather(xl, src2d, xr, dst2d)
    wvp, qc = _edge1(edge_attributes, gl, gr, We, w1c, att_row)
    acc = sc_scatter(wvp, dst2d, zrows)
    g1, g2 = _node2(acc[0], acc[1], bias_row, w1a, w1b)
    gg1, gg2 = sc_gather(g1, src2d, g2, dst2d)
    trust = _edge2(gg1, gg2, qc, b1_row, w2t, b2_row)
    return trust.reshape(E)


# speedup vs baseline: 23.7339x; 1.2382x over previous
import functools

import jax
import jax.numpy as jnp
from jax import lax
from jax.experimental import pallas as pl
from jax.experimental.pallas import tpu as pltpu
from jax.experimental.pallas import tpu_sc as plsc

N = 10000
E = 320000
D = 128
HH = 64
EDGE_DIM = 125
EMB = 124

NC, NS = 2, 16
NW = NC * NS
CH = 128
NCHUNK = E // CH
CPW = 80
PCH = NW * CPW
GRP = 2
NGRP = CPW // GRP
ROWS_PER_SUB = N // NS

EB = 2000
NB = 1000
ACC_W = 80


def _node_mm_body(x_ref, w_ref, xl_ref, xr_ref):
    o = jnp.dot(x_ref[...], w_ref[...], preferred_element_type=jnp.float32)
    xl_ref[...] = o[:, :HH]
    xr_ref[...] = o[:, HH:]


def _node_mm(x, w_node):
    return pl.pallas_call(
        _node_mm_body,
        grid=(N // NB,),
        in_specs=[
            pl.BlockSpec((NB, D), lambda i: (i, 0)),
            pl.BlockSpec((D, 2 * HH), lambda i: (0, 0)),
        ],
        out_specs=[pl.BlockSpec((NB, HH), lambda i: (i, 0))] * 2,
        out_shape=[jax.ShapeDtypeStruct((N, HH), jnp.float32)] * 2,
    )(x, w_node)


def _edge1_body(ea_ref, gl_ref, gr_ref, wcomb_ref, attm_ref, pexp_ref,
                wvp_ref, qc_ref):
    gl = gl_ref[...]
    mm = jnp.dot(ea_ref[...], wcomb_ref[...],
                 preferred_element_type=jnp.float32)
    m = gl + gr_ref[...] + mm[:, :HH]
    m = jnp.where(m >= 0, m, 0.2 * m)
    p = jnp.exp(jnp.dot(m, attm_ref[...],
                        preferred_element_type=jnp.float32))
    pbc = jnp.dot(p, pexp_ref[...],
                  preferred_element_type=jnp.float32)
    wvp_ref[...] = jnp.concatenate(
        [gl * pbc, p, jnp.zeros((EB, ACC_W - HH - 2), jnp.float32)], axis=1)
    qc_ref[...] = mm[:, HH:]


def _edge1(edge_attr, gl, gr, wcomb, attm, pexp):
    return pl.pallas_call(
        _edge1_body,
        grid=(E // EB,),
        in_specs=[
            pl.BlockSpec((EB, EDGE_DIM), lambda i: (i, 0)),
            pl.BlockSpec((EB, HH), lambda i: (i, 0)),
            pl.BlockSpec((EB, HH), lambda i: (i, 0)),
            pl.BlockSpec((EDGE_DIM, 2 * HH), lambda i: (0, 0)),
            pl.BlockSpec((HH, 2), lambda i: (0, 0)),
            pl.BlockSpec((2, HH), lambda i: (0, 0)),
        ],
        out_specs=[
            pl.BlockSpec((EB, ACC_W), lambda i: (i, 0)),
            pl.BlockSpec((EB, HH), lambda i: (i, 0)),
        ],
        out_shape=[
            jax.ShapeDtypeStruct((E, ACC_W), jnp.float32),
            jax.ShapeDtypeStruct((E, HH), jnp.float32),
        ],
    )(edge_attr, gl, gr, wcomb, attm, pexp)


def _node2_body(a0_ref, a1_ref, bias_ref, w1ab_ref, g1_ref, g2_ref):
    s = a0_ref[...] + a1_ref[...]
    v = s[:, :HH]
    d0 = s[:, HH:HH + 1] + 1e-16
    d1 = s[:, HH + 1:HH + 2] + 1e-16
    h = jnp.concatenate([v[:, :32] / d0, v[:, 32:] / d1], axis=1)
    h = jnp.maximum(h + bias_ref[...], 0.0)
    o = jnp.dot(h, w1ab_ref[...], preferred_element_type=jnp.float32)
    g1_ref[...] = o[:, :HH]
    g2_ref[...] = o[:, HH:]


def _node2(acc0, acc1, bias_row, w1ab):
    return pl.pallas_call(
        _node2_body,
        grid=(N // NB,),
        in_specs=[
            pl.BlockSpec((NB, ACC_W), lambda i: (i, 0)),
            pl.BlockSpec((NB, ACC_W), lambda i: (i, 0)),
            pl.BlockSpec((1, HH), lambda i: (0, 0)),
            pl.BlockSpec((HH, 2 * HH), lambda i: (0, 0)),
        ],
        out_specs=[pl.BlockSpec((NB, HH), lambda i: (i, 0))] * 2,
        out_shape=[jax.ShapeDtypeStruct((N, HH), jnp.float32)] * 2,
    )(acc0, acc1, bias_row, w1ab)


def _edge2_body(g1_ref, g2_ref, qc_ref, b1_ref, w2_ref, b2_ref,
                wnum_ref, wden_ref, out_ref):
    hcls = jnp.maximum(g1_ref[...] + g2_ref[...] + qc_ref[...] + b1_ref[...],
                       0.0)
    lg = jnp.dot(hcls, w2_ref[...],
                 preferred_element_type=jnp.float32) + b2_ref[...]
    e = jnp.exp(lg)
    num = jnp.dot(e, wnum_ref[...], preferred_element_type=jnp.float32)
    den = jnp.dot(e, wden_ref[...], preferred_element_type=jnp.float32)
    out_ref[...] = num / den


def _edge2(gg1, gg2, qc, b1_row, w2, b2_row, wnum, wden):
    return pl.pallas_call(
        _edge2_body,
        grid=(E // EB,),
        in_specs=[
            pl.BlockSpec((EB, HH), lambda i: (i, 0)),
            pl.BlockSpec((EB, HH), lambda i: (i, 0)),
            pl.BlockSpec((EB, HH), lambda i: (i, 0)),
            pl.BlockSpec((1, HH), lambda i: (0, 0)),
            pl.BlockSpec((HH, 3), lambda i: (0, 0)),
            pl.BlockSpec((1, 3), lambda i: (0, 0)),
            pl.BlockSpec((3, 1), lambda i: (0, 0)),
            pl.BlockSpec((3, 1), lambda i: (0, 0)),
        ],
        out_specs=pl.BlockSpec((EB, 1), lambda i: (i, 0)),
        out_shape=jax.ShapeDtypeStruct((E, 1), jnp.float32),
    )(gg1, gg2, qc, b1_row, w2, b2_row, wnum, wden)


def _sc_gather_body(tab_a, idx_a, tab_b, idx_b, out_a, out_b,
                    ia, ib, ra, rb, sg, sw0, sw1):
    w = lax.axis_index("s") * NC + lax.axis_index("c")
    base = w * CPW
    pltpu.sync_copy(idx_a.at[pl.ds(base, CPW)], ia)
    pltpu.sync_copy(idx_b.at[pl.ds(base, CPW)], ib)
    sws = (sw0, sw1)

    def real(g, k):
        return base + g * GRP + k < NCHUNK

    def g_descs(par, j, k):
        da = pltpu.make_async_copy(tab_a.at[ia.at[j]],
                                   ra.at[par, pl.ds(k * CH, CH)], sg)
        db = pltpu.make_async_copy(tab_b.at[ib.at[j]],
                                   rb.at[par, pl.ds(k * CH, CH)], sg)
        return da, db

    def wb_descs(par, g, k):
        row0 = (base + g * GRP + k) * CH
        da = pltpu.make_async_copy(ra.at[par, pl.ds(k * CH, CH)],
                                   out_a.at[pl.ds(row0, CH)], sws[par])
        db = pltpu.make_async_copy(rb.at[par, pl.ds(k * CH, CH)],
                                   out_b.at[pl.ds(row0, CH)], sws[par])
        return da, db

    def body(t, carry):
        for par in (0, 1):
            g = t * 2 + par
            for k in range(GRP):
                @pl.when((t > 0) & real(g - 2, k))
                def _(par=par, g=g, k=k):
                    da, db = wb_descs(par, g - 2, k)
                    da.wait()
                    db.wait()

                @pl.when(real(g, k))
                def _(par=par, g=g, k=k):
                    da, db = g_descs(par, g * GRP + k, k)
                    da.start()
                    db.start()
        for par in (0, 1):
            g = t * 2 + par
            for k in range(GRP):
                @pl.when(real(g, k))
                def _(par=par, g=g, k=k):
                    da, db = g_descs(par, g * GRP + k, k)
                    da.wait()
                    db.wait()
                    wa, wb_ = wb_descs(par, g, k)
                    wa.start()
                    wb_.start()
        return carry

    lax.fori_loop(0, NGRP // 2, body, 0)
    for par in (0, 1):
        g = NGRP - 2 + par
        for k in range(GRP):
            @pl.when(real(g, k))
            def _(par=par, g=g, k=k):
                da, db = wb_descs(par, g, k)
                da.wait()
                db.wait()


def _sc_scatter_body(wvp, idx2d, zrows, acc_out, acc_sh, id2, wb, sl0, sl1, ss):
    c = lax.axis_index("c")
    s = lax.axis_index("s")
    w = s * NC + c
    base = w * CPW
    pltpu.sync_copy(idx2d.at[pl.ds(base, CPW)], id2)
    pltpu.sync_copy(zrows.at[pl.ds(s * ROWS_PER_SUB, ROWS_PER_SUB)],
                    acc_sh.at[pl.ds(s * ROWS_PER_SUB, ROWS_PER_SUB)])
    plsc.subcore_barrier()
    sls = (sl0, sl1)

    def real(g, k):
        return base + g * GRP + k < NCHUNK

    def l_desc(par, g, k):
        row0 = (base + g * GRP + k) * CH
        return pltpu.make_async_copy(wvp.at[pl.ds(row0, CH)],
                                     wb.at[par, pl.ds(k * CH, CH)], sls[par])

    def s_desc(par, j, k):
        return pltpu.make_async_copy(wb.at[par, pl.ds(k * CH, CH)],
                                     acc_sh.at[id2.at[j]], ss)

    def body(t, carry):
        for par in (0, 1):
            g = t * 2 + par
            for k in range(GRP):
                @pl.when(real(g, k))
                def _(par=par, g=g, k=k):
                    l_desc(par, g, k).start()
        for par in (0, 1):
            g = t * 2 + par
            for k in range(GRP):
                @pl.when(real(g, k))
                def _(par=par, g=g, k=k):
                    l_desc(par, g, k).wait()
                    s_desc(par, g * GRP + k, k).start(add=True)
            for k in range(GRP):
                @pl.when(real(g, k))
                def _(par=par, g=g, k=k):
                    s_desc(par, g * GRP + k, k).wait()
        return carry

    lax.fori_loop(0, NGRP // 2, body, 0)
    plsc.subcore_barrier()
    pltpu.sync_copy(acc_sh.at[pl.ds(s * ROWS_PER_SUB, ROWS_PER_SUB)],
                    acc_out.at[c, pl.ds(s * ROWS_PER_SUB, ROWS_PER_SUB)])


@functools.cache
def _sc_kernels():
    mesh = plsc.VectorSubcoreMesh(core_axis_name="c", subcore_axis_name="s",
                                  num_cores=NC, num_subcores=NS)
    params = pltpu.CompilerParams(use_tc_tiling_on_sc=False)
    gather = pl.kernel(
        _sc_gather_body,
        out_type=[jax.ShapeDtypeStruct((E, HH), jnp.float32)] * 2,
        mesh=mesh,
        compiler_params=params,
        scratch_types=[
            pltpu.VMEM((CPW, CH), jnp.int32),
            pltpu.VMEM((CPW, CH), jnp.int32),
            pltpu.VMEM((2, GRP * CH, HH), jnp.float32),
            pltpu.VMEM((2, GRP * CH, HH), jnp.float32),
            pltpu.SemaphoreType.DMA,
            pltpu.SemaphoreType.DMA,
            pltpu.SemaphoreType.DMA,
        ],
    )
    scatter = pl.kernel(
        _sc_scatter_body,
        out_type=jax.ShapeDtypeStruct((NC, N, ACC_W), jnp.float32),
        mesh=mesh,
        compiler_params=params,
        scratch_types=[
            pltpu.VMEM_SHARED((N, ACC_W), jnp.float32),
            pltpu.VMEM((CPW, CH), jnp.int32),
            pltpu.VMEM((2, GRP * CH, ACC_W), jnp.float32),
            pltpu.SemaphoreType.DMA,
            pltpu.SemaphoreType.DMA,
            pltpu.SemaphoreType.DMA,
        ],
    )
    return gather, scatter


def kernel(x, edge_index, edge_attributes, Wl, Wr, We, att, bias,
           W1, b1, W2, b2):
    pad = jnp.zeros(((PCH - NCHUNK) * CH,), jnp.int32)
    src2d = jnp.concatenate(
        [edge_index[0].astype(jnp.int32), pad]).reshape(PCH, CH)
    dst2d = jnp.concatenate(
        [edge_index[1].astype(jnp.int32), pad]).reshape(PCH, CH)
    w_node = jnp.concatenate([Wl, Wr], axis=1)
    heads = (jnp.arange(HH) // 32)[:, None] == jnp.arange(2)[None, :]
    attm = jnp.where(heads, att.reshape(HH, 1), 0.0)
    pexp = heads.T.astype(jnp.float32)
    w1c_pad = jnp.concatenate(
        [W1[2 * HH:], jnp.zeros((1, HH), jnp.float32)], axis=0)
    wcomb = jnp.concatenate([We, w1c_pad], axis=1)
    w1ab = jnp.concatenate([W1[:HH], W1[HH:2 * HH]], axis=1)
    bias_row = bias.reshape(1, HH)
    b1_row = b1.reshape(1, HH)
    b2_row = b2.reshape(1, 3)
    wnum = jnp.array([[0.0], [0.5], [1.0]], jnp.float32)
    wden = jnp.ones((3, 1), jnp.float32)
    zrows = jnp.zeros((N, ACC_W), jnp.float32)

    sc_gather, sc_scatter = _sc_kernels()
    xl, xr = _node_mm(x, w_node)
    gl, gr = sc_gather(xl, src2d, xr, dst2d)
    wvp, qc = _edge1(edge_attributes, gl, gr, wcomb, attm, pexp)
    acc = sc_scatter(wvp, dst2d, zrows)
    g1, g2 = _node2(acc[0], acc[1], bias_row, w1ab)
    gg1, gg2 = sc_gather(g1, src2d, g2, dst2d)
    trust = _edge2(gg1, gg2, qc, b1_row, W2, b2_row, wnum, wden)
    return trust.reshape(E)


# speedup vs baseline: 26.3752x; 1.1113x over previous
import functools

import jax
import jax.numpy as jnp
from jax import lax
from jax.experimental import pallas as pl
from jax.experimental.pallas import tpu as pltpu
from jax.experimental.pallas import tpu_sc as plsc

N = 10000
E = 320000
D = 128
HH = 64
EDGE_DIM = 125
EMB = 124

NC, NS = 2, 16
NW = NC * NS
CH = 128
NCHUNK = E // CH
CPW = 80
PCH = NW * CPW
GRP = 2
NGRP = CPW // GRP
SGRP = 1
NGRP_S = CPW // SGRP
ROWS_PER_SUB = N // NS

EB = 2000
NB = 1000
ACC_W = 80


def _node_mm_body(x_ref, w_ref, xl_ref, xr_ref):
    o = jnp.dot(x_ref[...], w_ref[...], preferred_element_type=jnp.float32)
    xl_ref[...] = o[:, :HH]
    xr_ref[...] = o[:, HH:]


def _node_mm(x, w_node):
    return pl.pallas_call(
        _node_mm_body,
        grid=(N // NB,),
        in_specs=[
            pl.BlockSpec((NB, D), lambda i: (i, 0)),
            pl.BlockSpec((D, 2 * HH), lambda i: (0, 0)),
        ],
        out_specs=[pl.BlockSpec((NB, HH), lambda i: (i, 0))] * 2,
        out_shape=[jax.ShapeDtypeStruct((N, HH), jnp.float32)] * 2,
    )(x, w_node)


def _edge1_body(ea_ref, gs_ref, wcomb_ref, attm_ref, p32m_ref,
                p32_ref, qc_ref):
    mm = jnp.dot(ea_ref[...], wcomb_ref[...],
                 preferred_element_type=jnp.float32)
    m = gs_ref[...] + mm[:, :HH]
    m = jnp.where(m >= 0, m, 0.2 * m)
    p = jnp.exp(jnp.dot(m, attm_ref[...],
                        preferred_element_type=jnp.float32))
    p32_ref[...] = jnp.dot(p, p32m_ref[...],
                           preferred_element_type=jnp.float32)
    qc_ref[...] = mm[:, HH:].astype(jnp.bfloat16)


def _edge1(edge_attr, gs, wcomb, attm, p32m):
    return pl.pallas_call(
        _edge1_body,
        grid=(E // EB,),
        in_specs=[
            pl.BlockSpec((EB, EDGE_DIM), lambda i: (i, 0)),
            pl.BlockSpec((EB, HH), lambda i: (i, 0)),
            pl.BlockSpec((EDGE_DIM, 2 * HH), lambda i: (0, 0)),
            pl.BlockSpec((HH, 2), lambda i: (0, 0)),
            pl.BlockSpec((2, 32), lambda i: (0, 0)),
        ],
        out_specs=[
            pl.BlockSpec((EB, 32), lambda i: (i, 0)),
            pl.BlockSpec((EB, HH), lambda i: (i, 0)),
        ],
        out_shape=[
            jax.ShapeDtypeStruct((E, 32), jnp.float32),
            jax.ShapeDtypeStruct((E, HH), jnp.bfloat16),
        ],
    )(edge_attr, gs, wcomb, attm, p32m)


def _node2_body(a0_ref, a1_ref, bias_ref, w1ab_ref, g1_ref, g2_ref):
    s = a0_ref[...] + a1_ref[...]
    v = s[:, :HH]
    d0 = s[:, HH:HH + 1] + 1e-16
    d1 = s[:, HH + 1:HH + 2] + 1e-16
    h = jnp.concatenate([v[:, :32] / d0, v[:, 32:] / d1], axis=1)
    h = jnp.maximum(h + bias_ref[...], 0.0)
    o = jnp.dot(h, w1ab_ref[...], preferred_element_type=jnp.float32)
    g1_ref[...] = o[:, :HH].astype(jnp.bfloat16)
    g2_ref[...] = o[:, HH:].astype(jnp.bfloat16)


def _node2(acc0, acc1, bias_row, w1ab):
    return pl.pallas_call(
        _node2_body,
        grid=(N // NB,),
        in_specs=[
            pl.BlockSpec((NB, ACC_W), lambda i: (i, 0)),
            pl.BlockSpec((NB, ACC_W), lambda i: (i, 0)),
            pl.BlockSpec((1, HH), lambda i: (0, 0)),
            pl.BlockSpec((HH, 2 * HH), lambda i: (0, 0)),
        ],
        out_specs=[pl.BlockSpec((NB, HH), lambda i: (i, 0))] * 2,
        out_shape=[jax.ShapeDtypeStruct((N, HH), jnp.bfloat16)] * 2,
    )(acc0, acc1, bias_row, w1ab)


def _edge2_body(gs_ref, qc_ref, b1_ref, w2_ref, b2_ref,
                wnum_ref, wden_ref, out_ref):
    hcls = jnp.maximum(gs_ref[...].astype(jnp.float32)
                       + qc_ref[...].astype(jnp.float32) + b1_ref[...], 0.0)
    lg = jnp.dot(hcls, w2_ref[...],
                 preferred_element_type=jnp.float32) + b2_ref[...]
    e = jnp.exp(lg)
    num = jnp.dot(e, wnum_ref[...], preferred_element_type=jnp.float32)
    den = jnp.dot(e, wden_ref[...], preferred_element_type=jnp.float32)
    out_ref[...] = num / den


def _edge2(gs2, qc, b1_row, w2, b2_row, wnum, wden):
    return pl.pallas_call(
        _edge2_body,
        grid=(E // EB,),
        in_specs=[
            pl.BlockSpec((EB, HH), lambda i: (i, 0)),
            pl.BlockSpec((EB, HH), lambda i: (i, 0)),
            pl.BlockSpec((1, HH), lambda i: (0, 0)),
            pl.BlockSpec((HH, 3), lambda i: (0, 0)),
            pl.BlockSpec((1, 3), lambda i: (0, 0)),
            pl.BlockSpec((3, 1), lambda i: (0, 0)),
            pl.BlockSpec((3, 1), lambda i: (0, 0)),
        ],
        out_specs=pl.BlockSpec((EB, 1), lambda i: (i, 0)),
        out_shape=jax.ShapeDtypeStruct((E, 1), jnp.float32),
    )(gs2, qc, b1_row, w2, b2_row, wnum, wden)


def _sc_gather_sum_body(tab_a, idx_a, tab_b, idx_b, out, ia, ib, ra, rb,
                        sg, sw0, sw1, *, lanes):
    w = lax.axis_index("s") * NC + lax.axis_index("c")
    base = w * CPW
    pltpu.sync_copy(idx_a.at[pl.ds(base, CPW)], ia)
    pltpu.sync_copy(idx_b.at[pl.ds(base, CPW)], ib)
    sws = (sw0, sw1)

    def real(g, k):
        return base + g * GRP + k < NCHUNK

    def g_descs(par, j, k):
        da = pltpu.make_async_copy(tab_a.at[ia.at[j]],
                                   ra.at[par, pl.ds(k * CH, CH)], sg)
        db = pltpu.make_async_copy(tab_b.at[ib.at[j]],
                                   rb.at[par, pl.ds(k * CH, CH)], sg)
        return da, db

    def wb_desc(par, g, k):
        row0 = (base + g * GRP + k) * CH
        return pltpu.make_async_copy(ra.at[par, pl.ds(k * CH, CH)],
                                     out.at[pl.ds(row0, CH)], sws[par])

    def body(t, carry):
        for par in (0, 1):
            g = t * 2 + par
            for k in range(GRP):
                @pl.when((t > 0) & real(g - 2, k))
                def _(par=par, g=g, k=k):
                    wb_desc(par, g - 2, k).wait()

                @pl.when(real(g, k))
                def _(par=par, g=g, k=k):
                    da, db = g_descs(par, g * GRP + k, k)
                    da.start()
                    db.start()
        for par in (0, 1):
            g = t * 2 + par
            for k in range(GRP):
                @pl.when(real(g, k))
                def _(par=par, g=g, k=k):
                    da, db = g_descs(par, g * GRP + k, k)
                    da.wait()
                    db.wait()

            def addbody(r, carry2, par=par):
                for c in range(HH // lanes):
                    sl = pl.ds(c * lanes, lanes)
                    ra[par, r, sl] = ra[par, r, sl] + rb[par, r, sl]
                return carry2

            lax.fori_loop(0, GRP * CH, addbody, 0)
            for k in range(GRP):
                @pl.when(real(g, k))
                def _(par=par, g=g, k=k):
                    wb_desc(par, g, k).start()
        return carry

    lax.fori_loop(0, NGRP // 2, body, 0)
    for par in (0, 1):
        g = NGRP - 2 + par
        for k in range(GRP):
            @pl.when(real(g, k))
            def _(par=par, g=g, k=k):
                wb_desc(par, g, k).wait()


def _sc_scatter_body(p32, xl, sidx2d, didx2d, zrows, acc_out,
                     acc_sh, is2, id2, pb, xb, rows, sl0, sl1, ss):
    c = lax.axis_index("c")
    s = lax.axis_index("s")
    w = s * NC + c
    base = w * CPW
    pltpu.sync_copy(sidx2d.at[pl.ds(base, CPW)], is2)
    pltpu.sync_copy(didx2d.at[pl.ds(base, CPW)], id2)
    pltpu.sync_copy(zrows.at[pl.ds(s * ROWS_PER_SUB, ROWS_PER_SUB)],
                    acc_sh.at[pl.ds(s * ROWS_PER_SUB, ROWS_PER_SUB)])
    plsc.subcore_barrier()
    sls = (sl0, sl1)
    lane = lax.iota(jnp.int32, 16)

    def real(g, k):
        return base + g * SGRP + k < NCHUNK

    def l_descs(par, g, k):
        row0 = (base + g * SGRP + k) * CH
        dp = pltpu.make_async_copy(p32.at[pl.ds(row0, CH)],
                                   pb.at[par, pl.ds(k * CH, CH)], sls[par])
        dx = pltpu.make_async_copy(xl.at[is2.at[g * SGRP + k]],
                                   xb.at[par, pl.ds(k * CH, CH)], sls[par])
        return dp, dx

    def s_desc(par, j, k):
        return pltpu.make_async_copy(rows.at[par, pl.ds(k * CH, CH)],
                                     acc_sh.at[id2.at[j]], ss)

    def body(t, carry):
        for par in (0, 1):
            g = t * 2 + par
            for k in range(SGRP):
                @pl.when(real(g, k))
                def _(par=par, g=g, k=k):
                    dp, dx = l_descs(par, g, k)
                    dp.start()
                    dx.start()
        for par in (0, 1):
            g = t * 2 + par
            for k in range(SGRP):
                @pl.when(real(g, k))
                def _(par=par, g=g, k=k):
                    dp, dx = l_descs(par, g, k)
                    dp.wait()
                    dx.wait()

                    def build(e, carry2, par=par, k=k):
                        r = k * CH + e
                        p0 = pb[par, r, pl.ds(0, 16)]
                        p1 = pb[par, r, pl.ds(16, 16)]
                        for q in range(2):
                            sl = pl.ds(q * 16, 16)
                            rows[par, r, sl] = xb[par, r, sl] * p0
                        for q in range(2, 4):
                            sl = pl.ds(q * 16, 16)
                            rows[par, r, sl] = xb[par, r, sl] * p1
                        rows[par, r, pl.ds(HH, 16)] = jnp.where(
                            lane == 0, p0, jnp.where(lane == 1, p1, 0.0))
                        return carry2

                    lax.fori_loop(0, CH, build, 0)
                    s_desc(par, g * SGRP + k, k).start(add=True)
            for k in range(SGRP):
                @pl.when(real(g, k))
                def _(par=par, g=g, k=k):
                    s_desc(par, g * SGRP + k, k).wait()
        return carry

    lax.fori_loop(0, NGRP_S // 2, body, 0)
    plsc.subcore_barrier()
    pltpu.sync_copy(acc_sh.at[pl.ds(s * ROWS_PER_SUB, ROWS_PER_SUB)],
                    acc_out.at[c, pl.ds(s * ROWS_PER_SUB, ROWS_PER_SUB)])


@functools.cache
def _sc_kernels():
    mesh = plsc.VectorSubcoreMesh(core_axis_name="c", subcore_axis_name="s",
                                  num_cores=NC, num_subcores=NS)
    params = pltpu.CompilerParams(use_tc_tiling_on_sc=False)
    def make_gather_sum(dtype, lanes):
        return pl.kernel(
            functools.partial(_sc_gather_sum_body, lanes=lanes),
            out_type=jax.ShapeDtypeStruct((E, HH), dtype),
            mesh=mesh,
            compiler_params=params,
            scratch_types=[
                pltpu.VMEM((CPW, CH), jnp.int32),
                pltpu.VMEM((CPW, CH), jnp.int32),
                pltpu.VMEM((2, GRP * CH, HH), dtype),
                pltpu.VMEM((2, GRP * CH, HH), dtype),
                pltpu.SemaphoreType.DMA,
                pltpu.SemaphoreType.DMA,
                pltpu.SemaphoreType.DMA,
            ],
        )

    gather_sum_f32 = make_gather_sum(jnp.float32, 16)
    gather_sum_bf16 = make_gather_sum(jnp.bfloat16, 32)
    scatter = pl.kernel(
        _sc_scatter_body,
        out_type=jax.ShapeDtypeStruct((NC, N, ACC_W), jnp.float32),
        mesh=mesh,
        compiler_params=params,
        scratch_types=[
            pltpu.VMEM_SHARED((N, ACC_W), jnp.float32),
            pltpu.VMEM((CPW, CH), jnp.int32),
            pltpu.VMEM((CPW, CH), jnp.int32),
            pltpu.VMEM((2, SGRP * CH, 32), jnp.float32),
            pltpu.VMEM((2, SGRP * CH, HH), jnp.float32),
            pltpu.VMEM((2, SGRP * CH, ACC_W), jnp.float32),
            pltpu.SemaphoreType.DMA,
            pltpu.SemaphoreType.DMA,
            pltpu.SemaphoreType.DMA,
        ],
    )
    return gather_sum_f32, gather_sum_bf16, scatter


def kernel(x, edge_index, edge_attributes, Wl, Wr, We, att, bias,
           W1, b1, W2, b2):
    pad = jnp.zeros(((PCH - NCHUNK) * CH,), jnp.int32)
    src2d = jnp.concatenate(
        [edge_index[0].astype(jnp.int32), pad]).reshape(PCH, CH)
    dst2d = jnp.concatenate(
        [edge_index[1].astype(jnp.int32), pad]).reshape(PCH, CH)
    w_node = jnp.concatenate([Wl, Wr], axis=1)
    heads = (jnp.arange(HH) // 32)[:, None] == jnp.arange(2)[None, :]
    attm = jnp.where(heads, att.reshape(HH, 1), 0.0)
    p32m = ((jnp.arange(32) // 16)[None, :]
            == jnp.arange(2)[:, None]).astype(jnp.float32)
    w1c_pad = jnp.concatenate(
        [W1[2 * HH:], jnp.zeros((1, HH), jnp.float32)], axis=0)
    wcomb = jnp.concatenate([We, w1c_pad], axis=1)
    w1ab = jnp.concatenate([W1[:HH], W1[HH:2 * HH]], axis=1)
    bias_row = bias.reshape(1, HH)
    b1_row = b1.reshape(1, HH)
    b2_row = b2.reshape(1, 3)
    wnum = jnp.array([[0.0], [0.5], [1.0]], jnp.float32)
    wden = jnp.ones((3, 1), jnp.float32)
    zrows = jnp.zeros((N, ACC_W), jnp.float32)

    gather_sum_f32, gather_sum_bf16, sc_scatter = _sc_kernels()
    xl, xr = _node_mm(x, w_node)
    gs1 = gather_sum_f32(xl, src2d, xr, dst2d)
    p32, qc = _edge1(edge_attributes, gs1, wcomb, attm, p32m)
    acc = sc_scatter(p32, xl, src2d, dst2d, zrows)
    g1, g2 = _node2(acc[0], acc[1], bias_row, w1ab)
    gs2 = gather_sum_bf16(g1, src2d, g2, dst2d)
    trust = _edge2(gs2, qc, b1_row, W2, b2_row, wnum, wden)
    return trust.reshape(E)


# speedup vs baseline: 26.7495x; 1.0142x over previous
import functools

import jax
import jax.numpy as jnp
from jax import lax
from jax.experimental import pallas as pl
from jax.experimental.pallas import tpu as pltpu
from jax.experimental.pallas import tpu_sc as plsc

N = 10000
E = 320000
D = 128
HH = 64
EDGE_DIM = 125
EMB = 124

NC, NS = 2, 16
NW = NC * NS
CH = 128
NCHUNK = E // CH
CPW = 80
PCH = NW * CPW
GRP = 2
NGRP = CPW // GRP
SGRP = 1
NGRP_S = CPW // SGRP
ROWS_PER_SUB = N // NS

EB = 2000
NB = 1000
ACC_W = 80


def _node_mm_body(x_ref, w_ref, xl_ref, xr_ref):
    o = jnp.dot(x_ref[...], w_ref[...], preferred_element_type=jnp.float32)
    xl_ref[...] = o[:, :HH]
    xr_ref[...] = o[:, HH:]


def _node_mm(x, w_node):
    return pl.pallas_call(
        _node_mm_body,
        grid=(N // NB,),
        in_specs=[
            pl.BlockSpec((NB, D), lambda i: (i, 0)),
            pl.BlockSpec((D, 2 * HH), lambda i: (0, 0)),
        ],
        out_specs=[pl.BlockSpec((NB, HH), lambda i: (i, 0))] * 2,
        out_shape=[jax.ShapeDtypeStruct((N, HH), jnp.float32)] * 2,
    )(x, w_node)


def _edge1_body(ea_ref, gs_ref, wcomb_ref, attm_ref, p32m_ref,
                p32_ref, qc_ref):
    mm = jnp.dot(ea_ref[...], wcomb_ref[...],
                 preferred_element_type=jnp.float32)
    m = gs_ref[...] + mm[:, :HH]
    m = jnp.where(m >= 0, m, 0.2 * m)
    p = jnp.exp(jnp.dot(m, attm_ref[...],
                        preferred_element_type=jnp.float32))
    p32_ref[...] = jnp.dot(p, p32m_ref[...],
                           preferred_element_type=jnp.float32)
    qc_ref[...] = mm[:, HH:]


def _edge1(edge_attr, gs, wcomb, attm, p32m):
    return pl.pallas_call(
        _edge1_body,
        grid=(E // EB,),
        in_specs=[
            pl.BlockSpec((EB, EDGE_DIM), lambda i: (i, 0)),
            pl.BlockSpec((EB, HH), lambda i: (i, 0)),
            pl.BlockSpec((EDGE_DIM, 2 * HH), lambda i: (0, 0)),
            pl.BlockSpec((HH, 2), lambda i: (0, 0)),
            pl.BlockSpec((2, 32), lambda i: (0, 0)),
        ],
        out_specs=[
            pl.BlockSpec((EB, 32), lambda i: (i, 0)),
            pl.BlockSpec((EB, HH), lambda i: (i, 0)),
        ],
        out_shape=[
            jax.ShapeDtypeStruct((E, 32), jnp.float32),
            jax.ShapeDtypeStruct((E, HH), jnp.float32),
        ],
    )(edge_attr, gs, wcomb, attm, p32m)


def _node2_body(a0_ref, a1_ref, bias_ref, w1ab_ref, g1_ref, g2_ref):
    s = a0_ref[...] + a1_ref[...]
    v = s[:, :HH]
    d0 = s[:, HH:HH + 1] + 1e-16
    d1 = s[:, HH + 1:HH + 2] + 1e-16
    h = jnp.concatenate([v[:, :32] / d0, v[:, 32:] / d1], axis=1)
    h = jnp.maximum(h + bias_ref[...], 0.0)
    o = jnp.dot(h, w1ab_ref[...], preferred_element_type=jnp.float32)
    g1_ref[...] = o[:, :HH]
    g2_ref[...] = o[:, HH:]


def _node2(acc0, acc1, bias_row, w1ab):
    return pl.pallas_call(
        _node2_body,
        grid=(N // NB,),
        in_specs=[
            pl.BlockSpec((NB, ACC_W), lambda i: (i, 0)),
            pl.BlockSpec((NB, ACC_W), lambda i: (i, 0)),
            pl.BlockSpec((1, HH), lambda i: (0, 0)),
            pl.BlockSpec((HH, 2 * HH), lambda i: (0, 0)),
        ],
        out_specs=[pl.BlockSpec((NB, HH), lambda i: (i, 0))] * 2,
        out_shape=[jax.ShapeDtypeStruct((N, HH), jnp.float32)] * 2,
    )(acc0, acc1, bias_row, w1ab)


def _edge2_body(gs_ref, qc_ref, b1_ref, w2_ref, b2_ref,
                wnum_ref, wden_ref, out_ref):
    hcls = jnp.maximum(gs_ref[...] + qc_ref[...] + b1_ref[...], 0.0)
    lg = jnp.dot(hcls, w2_ref[...],
                 preferred_element_type=jnp.float32) + b2_ref[...]
    e = jnp.exp(lg)
    num = jnp.dot(e, wnum_ref[...], preferred_element_type=jnp.float32)
    den = jnp.dot(e, wden_ref[...], preferred_element_type=jnp.float32)
    out_ref[...] = num / den


def _edge2(gs2, qc, b1_row, w2, b2_row, wnum, wden):
    return pl.pallas_call(
        _edge2_body,
        grid=(E // EB,),
        in_specs=[
            pl.BlockSpec((EB, HH), lambda i: (i, 0)),
            pl.BlockSpec((EB, HH), lambda i: (i, 0)),
            pl.BlockSpec((1, HH), lambda i: (0, 0)),
            pl.BlockSpec((HH, 3), lambda i: (0, 0)),
            pl.BlockSpec((1, 3), lambda i: (0, 0)),
            pl.BlockSpec((3, 1), lambda i: (0, 0)),
            pl.BlockSpec((3, 1), lambda i: (0, 0)),
        ],
        out_specs=pl.BlockSpec((EB, 1), lambda i: (i, 0)),
        out_shape=jax.ShapeDtypeStruct((E, 1), jnp.float32),
    )(gs2, qc, b1_row, w2, b2_row, wnum, wden)


def _sc_gather_sum_body(tab_a, idx_a, tab_b, idx_b, out, ia, ib, ra, rb,
                        sg, sw0, sw1):
    w = lax.axis_index("s") * NC + lax.axis_index("c")
    base = w * CPW
    pltpu.sync_copy(idx_a.at[pl.ds(base, CPW)], ia)
    pltpu.sync_copy(idx_b.at[pl.ds(base, CPW)], ib)
    sws = (sw0, sw1)

    def real(g, k):
        return base + g * GRP + k < NCHUNK

    def g_descs(par, j, k):
        da = pltpu.make_async_copy(tab_a.at[ia.at[j]],
                                   ra.at[par, pl.ds(k * CH, CH)], sg)
        db = pltpu.make_async_copy(tab_b.at[ib.at[j]],
                                   rb.at[par, pl.ds(k * CH, CH)], sg)
        return da, db

    def wb_desc(par, g, k):
        row0 = (base + g * GRP + k) * CH
        return pltpu.make_async_copy(ra.at[par, pl.ds(k * CH, CH)],
                                     out.at[pl.ds(row0, CH)], sws[par])

    def body(t, carry):
        for par in (0, 1):
            g = t * 2 + par
            for k in range(GRP):
                @pl.when((t > 0) & real(g - 2, k))
                def _(par=par, g=g, k=k):
                    wb_desc(par, g - 2, k).wait()

                @pl.when(real(g, k))
                def _(par=par, g=g, k=k):
                    da, db = g_descs(par, g * GRP + k, k)
                    da.start()
                    db.start()
        for par in (0, 1):
            g = t * 2 + par
            for k in range(GRP):
                @pl.when(real(g, k))
                def _(par=par, g=g, k=k):
                    da, db = g_descs(par, g * GRP + k, k)
                    da.wait()
                    db.wait()

            def addbody(r, carry2, par=par):
                for c in range(HH // 16):
                    sl = pl.ds(c * 16, 16)
                    ra[par, r, sl] = ra[par, r, sl] + rb[par, r, sl]
                return carry2

            lax.fori_loop(0, GRP * CH, addbody, 0)
            for k in range(GRP):
                @pl.when(real(g, k))
                def _(par=par, g=g, k=k):
                    wb_desc(par, g, k).start()
        return carry

    lax.fori_loop(0, NGRP // 2, body, 0)
    for par in (0, 1):
        g = NGRP - 2 + par
        for k in range(GRP):
            @pl.when(real(g, k))
            def _(par=par, g=g, k=k):
                wb_desc(par, g, k).wait()


def _sc_scatter_body(p32, xl, sidx2d, didx2d, zrows, acc_out,
                     acc_sh, is2, id2, pb, xb, rows, sl0, sl1, ss):
    c = lax.axis_index("c")
    s = lax.axis_index("s")
    w = s * NC + c
    base = w * CPW
    pltpu.sync_copy(sidx2d.at[pl.ds(base, CPW)], is2)
    pltpu.sync_copy(didx2d.at[pl.ds(base, CPW)], id2)
    pltpu.sync_copy(zrows.at[pl.ds(s * ROWS_PER_SUB, ROWS_PER_SUB)],
                    acc_sh.at[pl.ds(s * ROWS_PER_SUB, ROWS_PER_SUB)])
    plsc.subcore_barrier()
    sls = (sl0, sl1)
    lane = lax.iota(jnp.int32, 16)

    def real(g, k):
        return base + g * SGRP + k < NCHUNK

    def l_descs(par, g, k):
        row0 = (base + g * SGRP + k) * CH
        dp = pltpu.make_async_copy(p32.at[pl.ds(row0, CH)],
                                   pb.at[par, pl.ds(k * CH, CH)], sls[par])
        dx = pltpu.make_async_copy(xl.at[is2.at[g * SGRP + k]],
                                   xb.at[par, pl.ds(k * CH, CH)], sls[par])
        return dp, dx

    def s_desc(par, j, k):
        return pltpu.make_async_copy(rows.at[par, pl.ds(k * CH, CH)],
                                     acc_sh.at[id2.at[j]], ss)

    def body(t, carry):
        for par in (0, 1):
            g = t * 2 + par
            for k in range(SGRP):
                @pl.when(real(g, k))
                def _(par=par, g=g, k=k):
                    dp, dx = l_descs(par, g, k)
                    dp.start()
                    dx.start()
        for par in (0, 1):
            g = t * 2 + par
            for k in range(SGRP):
                @pl.when(real(g, k))
                def _(par=par, g=g, k=k):
                    dp, dx = l_descs(par, g, k)
                    dp.wait()
                    dx.wait()

                    def build(e, carry2, par=par, k=k):
                        r = k * CH + e
                        p0 = pb[par, r, pl.ds(0, 16)]
                        p1 = pb[par, r, pl.ds(16, 16)]
                        for q in range(2):
                            sl = pl.ds(q * 16, 16)
                            rows[par, r, sl] = xb[par, r, sl] * p0
                        for q in range(2, 4):
                            sl = pl.ds(q * 16, 16)
                            rows[par, r, sl] = xb[par, r, sl] * p1
                        rows[par, r, pl.ds(HH, 16)] = jnp.where(
                            lane == 0, p0, jnp.where(lane == 1, p1, 0.0))
                        return carry2

                    lax.fori_loop(0, CH, build, 0)
                    s_desc(par, g * SGRP + k, k).start(add=True)
            for k in range(SGRP):
                @pl.when(real(g, k))
                def _(par=par, g=g, k=k):
                    s_desc(par, g * SGRP + k, k).wait()
        return carry

    lax.fori_loop(0, NGRP_S // 2, body, 0)
    plsc.subcore_barrier()
    pltpu.sync_copy(acc_sh.at[pl.ds(s * ROWS_PER_SUB, ROWS_PER_SUB)],
                    acc_out.at[c, pl.ds(s * ROWS_PER_SUB, ROWS_PER_SUB)])


@functools.cache
def _sc_kernels():
    mesh = plsc.VectorSubcoreMesh(core_axis_name="c", subcore_axis_name="s",
                                  num_cores=NC, num_subcores=NS)
    params = pltpu.CompilerParams(use_tc_tiling_on_sc=False)
    gather_sum = pl.kernel(
        _sc_gather_sum_body,
        out_type=jax.ShapeDtypeStruct((E, HH), jnp.float32),
        mesh=mesh,
        compiler_params=params,
        scratch_types=[
            pltpu.VMEM((CPW, CH), jnp.int32),
            pltpu.VMEM((CPW, CH), jnp.int32),
            pltpu.VMEM((2, GRP * CH, HH), jnp.float32),
            pltpu.VMEM((2, GRP * CH, HH), jnp.float32),
            pltpu.SemaphoreType.DMA,
            pltpu.SemaphoreType.DMA,
            pltpu.SemaphoreType.DMA,
        ],
    )
    scatter = pl.kernel(
        _sc_scatter_body,
        out_type=jax.ShapeDtypeStruct((NC, N, ACC_W), jnp.float32),
        mesh=mesh,
        compiler_params=params,
        scratch_types=[
            pltpu.VMEM_SHARED((N, ACC_W), jnp.float32),
            pltpu.VMEM((CPW, CH), jnp.int32),
            pltpu.VMEM((CPW, CH), jnp.int32),
            pltpu.VMEM((2, SGRP * CH, 32), jnp.float32),
            pltpu.VMEM((2, SGRP * CH, HH), jnp.float32),
            pltpu.VMEM((2, SGRP * CH, ACC_W), jnp.float32),
            pltpu.SemaphoreType.DMA,
            pltpu.SemaphoreType.DMA,
            pltpu.SemaphoreType.DMA,
        ],
    )
    return gather_sum, scatter


def kernel(x, edge_index, edge_attributes, Wl, Wr, We, att, bias,
           W1, b1, W2, b2):
    pad = jnp.zeros(((PCH - NCHUNK) * CH,), jnp.int32)
    src2d = jnp.concatenate(
        [edge_index[0].astype(jnp.int32), pad]).reshape(PCH, CH)
    dst2d = jnp.concatenate(
        [edge_index[1].astype(jnp.int32), pad]).reshape(PCH, CH)
    w_node = jnp.concatenate([Wl, Wr], axis=1)
    heads = (jnp.arange(HH) // 32)[:, None] == jnp.arange(2)[None, :]
    attm = jnp.where(heads, att.reshape(HH, 1), 0.0)
    p32m = ((jnp.arange(32) // 16)[None, :]
            == jnp.arange(2)[:, None]).astype(jnp.float32)
    w1c_pad = jnp.concatenate(
        [W1[2 * HH:], jnp.zeros((1, HH), jnp.float32)], axis=0)
    wcomb = jnp.concatenate([We, w1c_pad], axis=1)
    w1ab = jnp.concatenate([W1[:HH], W1[HH:2 * HH]], axis=1)
    bias_row = bias.reshape(1, HH)
    b1_row = b1.reshape(1, HH)
    b2_row = b2.reshape(1, 3)
    wnum = jnp.array([[0.0], [0.5], [1.0]], jnp.float32)
    wden = jnp.ones((3, 1), jnp.float32)
    zrows = jnp.zeros((N, ACC_W), jnp.float32)

    sc_gather_sum, sc_scatter = _sc_kernels()
    xl, xr = _node_mm(x, w_node)
    gs1 = sc_gather_sum(xl, src2d, xr, dst2d)
    p32, qc = _edge1(edge_attributes, gs1, wcomb, attm, p32m)
    acc = sc_scatter(p32, xl, src2d, dst2d, zrows)
    g1, g2 = _node2(acc[0], acc[1], bias_row, w1ab)
    gs2 = sc_gather_sum(g1, src2d, g2, dst2d)
    trust = _edge2(gs2, qc, b1_row, W2, b2_row, wnum, wden)
    return trust.reshape(E)
